# Initial kernel scaffold; baseline (speedup 1.0000x reference)
import jax
import jax.numpy as jnp
from jax.experimental import pallas as pl


def kernel(x, edge_index, W1, b1, W2, b2):
    raise NotImplementedError("write your pallas kernel here")


# speedup vs baseline: 11.2295x; 11.2295x over previous
import functools

import jax
import jax.numpy as jnp
from jax import lax
from jax.experimental import pallas as pl
from jax.experimental.pallas import tpu as pltpu
from jax.experimental.pallas import tpu_sc as plsc

N_NODES = 10000
NPAD = 10240
IN_CH = 128
HID_CH = 256
OUT_CH = 128
N_EDGES = 320000

NC = 2
NS = 16
SLAB = NPAD // NS

CH = 80
NCH_DEG = N_EDGES // (NC * NS * CH)
NCH_AGG = N_EDGES // (NS * CH)

_MESH = dict(core_axis_name="c", subcore_axis_name="s")


def _deg_call(dst32):
    @functools.partial(
        pl.kernel,
        out_type=jax.ShapeDtypeStruct((NC, NPAD), jnp.float32),
        mesh=plsc.VectorSubcoreMesh(**_MESH),
        scratch_types=[
            pltpu.VMEM((NCH_DEG, CH), jnp.int32),
            pltpu.VMEM((CH,), jnp.float32),
            pltpu.VMEM((SLAB,), jnp.float32),
            pltpu.VMEM_SHARED((NPAD,), jnp.float32),
        ],
        compiler_params=pltpu.CompilerParams(use_tc_tiling_on_sc=False),
    )
    def deg_k(dst_hbm, out_hbm, idx_v, ones_v, z_v, acc_sh):
        cid = lax.axis_index("c")
        sid = lax.axis_index("s")
        w = cid * NS + sid
        for i in range(CH // 16):
            ones_v[pl.ds(i * 16, 16)] = jnp.ones((16,), jnp.float32)
        for i in range(SLAB // 16):
            z_v[pl.ds(i * 16, 16)] = jnp.zeros((16,), jnp.float32)
        pltpu.sync_copy(z_v, acc_sh.at[pl.ds(sid * SLAB, SLAB)])
        pltpu.sync_copy(dst_hbm.at[w], idx_v)
        plsc.subcore_barrier()

        def body(j, carry):
            pltpu.sync_copy(ones_v, acc_sh.at[idx_v.at[j]], add=True)
            return carry

        lax.fori_loop(0, NCH_DEG, body, 0)
        plsc.subcore_barrier()
        pltpu.sync_copy(acc_sh.at[pl.ds(sid * SLAB, SLAB)],
                        out_hbm.at[cid, pl.ds(sid * SLAB, SLAB)])

    return deg_k(dst32)


def _agg_call(ya, yb, src16, dst16, d):
    @functools.partial(
        pl.kernel,
        out_type=jax.ShapeDtypeStruct((NC, NPAD, d), jnp.float32),
        mesh=plsc.VectorSubcoreMesh(**_MESH),
        scratch_types=[
            pltpu.VMEM((NCH_AGG, CH), jnp.int32),
            pltpu.VMEM((NCH_AGG, CH), jnp.int32),
            pltpu.VMEM((CH, d), jnp.float32),
            pltpu.VMEM_SHARED((NPAD, d), jnp.float32),
            pltpu.SemaphoreType.DMA,
        ],
        compiler_params=pltpu.CompilerParams(use_tc_tiling_on_sc=False),
    )
    def agg_k(ya_hbm, yb_hbm, src_hbm, dst_hbm, out_hbm,
              src_v, dst_v, rows_v, acc_sh, sem):
        cid = lax.axis_index("c")
        sid = lax.axis_index("s")
        pltpu.sync_copy(src_hbm.at[sid], src_v)
        pltpu.sync_copy(dst_hbm.at[sid], dst_v)

        @pl.when(cid == 0)
        def _():
            pltpu.sync_copy(ya_hbm.at[pl.ds(sid * SLAB, SLAB)],
                            acc_sh.at[pl.ds(sid * SLAB, SLAB)])

        @pl.when(cid == 1)
        def _():
            pltpu.sync_copy(yb_hbm.at[pl.ds(sid * SLAB, SLAB)],
                            acc_sh.at[pl.ds(sid * SLAB, SLAB)])

        plsc.subcore_barrier()

        def body(j, carry):
            @pl.when(cid == 0)
            def _():
                pltpu.async_copy(ya_hbm.at[src_v.at[j]], rows_v, sem).wait()

            @pl.when(cid == 1)
            def _():
                pltpu.async_copy(yb_hbm.at[src_v.at[j]], rows_v, sem).wait()

            pltpu.sync_copy(rows_v, acc_sh.at[dst_v.at[j]], add=True)
            return carry

        lax.fori_loop(0, NCH_AGG, body, 0)
        plsc.subcore_barrier()
        pltpu.sync_copy(acc_sh.at[pl.ds(sid * SLAB, SLAB)],
                        out_hbm.at[cid, pl.ds(sid * SLAB, SLAB)])

    return agg_k(ya, yb, src16, dst16)


_R = 512


def _mm1_call(xpad, W1, deg):
    Q = HID_CH // 4

    def body(x_ref, w_ref, deg_ref, q0_ref, q1_ref, q2_ref, q3_ref, dinv_ref):
        d = deg_ref[0] + deg_ref[1] + 1.0
        dinv = lax.rsqrt(d)
        y = jnp.dot(x_ref[...], w_ref[...],
                    preferred_element_type=jnp.float32) * dinv
        q0_ref[...] = y[:, 0 * Q: 1 * Q]
        q1_ref[...] = y[:, 1 * Q: 2 * Q]
        q2_ref[...] = y[:, 2 * Q: 3 * Q]
        q3_ref[...] = y[:, 3 * Q: 4 * Q]
        dinv_ref[...] = dinv

    qspec = pl.BlockSpec((_R, Q), lambda i: (i, 0))
    qshape = jax.ShapeDtypeStruct((NPAD, Q), jnp.float32)
    return pl.pallas_call(
        body,
        grid=(NPAD // _R,),
        in_specs=[
            pl.BlockSpec((_R, IN_CH), lambda i: (i, 0)),
            pl.BlockSpec((IN_CH, HID_CH), lambda i: (0, 0)),
            pl.BlockSpec((NC, _R, 1), lambda i: (0, i, 0)),
        ],
        out_specs=[qspec, qspec, qspec, qspec,
                   pl.BlockSpec((_R, 1), lambda i: (i, 0))],
        out_shape=[qshape, qshape, qshape, qshape,
                   jax.ShapeDtypeStruct((NPAD, 1), jnp.float32)],
    )(xpad, W1, deg)


def _mm2_call(a0, a1, a2, a3, dinv, b1r, W2):
    Q = HID_CH // 4

    def body(a0_ref, a1_ref, a2_ref, a3_ref, dinv_ref, b1_ref, w_ref,
             ya_ref, yb_ref):
        dinv = dinv_ref[...]
        acc = jnp.concatenate(
            [a0_ref[...], a1_ref[...], a2_ref[...], a3_ref[...]], axis=1)
        h = jax.nn.relu(acc * dinv + b1_ref[...])
        y2 = jnp.dot(h, w_ref[...], preferred_element_type=jnp.float32) * dinv
        ya_ref[...] = y2[:, : OUT_CH // 2]
        yb_ref[...] = y2[:, OUT_CH // 2:]

    qspec = pl.BlockSpec((_R, Q), lambda i: (i, 0))
    return pl.pallas_call(
        body,
        grid=(NPAD // _R,),
        in_specs=[
            qspec, qspec, qspec, qspec,
            pl.BlockSpec((_R, 1), lambda i: (i, 0)),
            pl.BlockSpec((1, HID_CH), lambda i: (0, 0)),
            pl.BlockSpec((HID_CH, OUT_CH), lambda i: (0, 0)),
        ],
        out_specs=[
            pl.BlockSpec((_R, OUT_CH // 2), lambda i: (i, 0)),
            pl.BlockSpec((_R, OUT_CH // 2), lambda i: (i, 0)),
        ],
        out_shape=[
            jax.ShapeDtypeStruct((NPAD, OUT_CH // 2), jnp.float32),
            jax.ShapeDtypeStruct((NPAD, OUT_CH // 2), jnp.float32),
        ],
    )(a0, a1, a2, a3, dinv, b1r, W2)


def _fin_call(acc_a, acc_b, dinv, b2a, b2b):
    def body(aa_ref, ab_ref, dinv_ref, b2a_ref, b2b_ref, out_ref):
        dinv = dinv_ref[...]
        o1 = aa_ref[...] * dinv + b2a_ref[...]
        o2 = ab_ref[...] * dinv + b2b_ref[...]
        out_ref[...] = jnp.concatenate([o1, o2], axis=1)

    O2 = OUT_CH // 2
    return pl.pallas_call(
        body,
        grid=(NPAD // _R,),
        in_specs=[
            pl.BlockSpec((_R, O2), lambda i: (i, 0)),
            pl.BlockSpec((_R, O2), lambda i: (i, 0)),
            pl.BlockSpec((_R, 1), lambda i: (i, 0)),
            pl.BlockSpec((1, O2), lambda i: (0, 0)),
            pl.BlockSpec((1, O2), lambda i: (0, 0)),
        ],
        out_specs=pl.BlockSpec((_R, OUT_CH), lambda i: (i, 0)),
        out_shape=jax.ShapeDtypeStruct((NPAD, OUT_CH), jnp.float32),
    )(acc_a, acc_b, dinv, b2a, b2b)


def kernel(x, edge_index, W1, b1, W2, b2):
    ei = edge_index.astype(jnp.int32)
    src, dst = ei[0], ei[1]
    dst32 = dst.reshape(NC * NS, NCH_DEG, CH)
    src16 = src.reshape(NS, NCH_AGG, CH)
    dst16 = dst.reshape(NS, NCH_AGG, CH)
    xpad = jnp.pad(x, ((0, NPAD - N_NODES), (0, 0)))

    deg = _deg_call(dst32)
    q0, q1, q2, q3, dinv = _mm1_call(xpad, W1, deg.reshape(NC, NPAD, 1))
    accA = _agg_call(q0, q1, src16, dst16, HID_CH // 4)
    accB = _agg_call(q2, q3, src16, dst16, HID_CH // 4)
    y2a, y2b = _mm2_call(accA[0], accA[1], accB[0], accB[1], dinv,
                         b1.reshape(1, -1), W2)
    acc2 = _agg_call(y2a, y2b, src16, dst16, OUT_CH // 2)
    out = _fin_call(acc2[0], acc2[1], dinv,
                    b2[: OUT_CH // 2].reshape(1, -1),
                    b2[OUT_CH // 2:].reshape(1, -1))
    return out[:N_NODES]


# speedup vs baseline: 17.9795x; 1.6011x over previous
import functools

import jax
import jax.numpy as jnp
from jax import lax
from jax.experimental import pallas as pl
from jax.experimental.pallas import tpu as pltpu
from jax.experimental.pallas import tpu_sc as plsc

N_NODES = 10000
NPAD = 10240
IN_CH = 128
HID_CH = 256
OUT_CH = 128
N_EDGES = 320000

NC = 2
NS = 16
SLAB = NPAD // NS

CH = 80
NCH_DEG = N_EDGES // (NC * NS * CH)
NCH_AGG = N_EDGES // (NS * CH)

_MESH = dict(core_axis_name="c", subcore_axis_name="s")


def _deg_call(dst32):
    @functools.partial(
        pl.kernel,
        out_type=jax.ShapeDtypeStruct((NC, NPAD), jnp.float32),
        mesh=plsc.VectorSubcoreMesh(**_MESH),
        scratch_types=[
            pltpu.VMEM((NCH_DEG, CH), jnp.int32),
            pltpu.VMEM((CH,), jnp.float32),
            pltpu.VMEM((SLAB,), jnp.float32),
            pltpu.VMEM_SHARED((NPAD,), jnp.float32),
        ],
        compiler_params=pltpu.CompilerParams(use_tc_tiling_on_sc=False),
    )
    def deg_k(dst_hbm, out_hbm, idx_v, ones_v, z_v, acc_sh):
        cid = lax.axis_index("c")
        sid = lax.axis_index("s")
        w = cid * NS + sid
        for i in range(CH // 16):
            ones_v[pl.ds(i * 16, 16)] = jnp.ones((16,), jnp.float32)
        for i in range(SLAB // 16):
            z_v[pl.ds(i * 16, 16)] = jnp.zeros((16,), jnp.float32)
        pltpu.sync_copy(z_v, acc_sh.at[pl.ds(sid * SLAB, SLAB)])
        pltpu.sync_copy(dst_hbm.at[w], idx_v)
        plsc.subcore_barrier()

        def body(j, carry):
            pltpu.sync_copy(ones_v, acc_sh.at[idx_v.at[j]], add=True)
            return carry

        lax.fori_loop(0, NCH_DEG, body, 0)
        plsc.subcore_barrier()
        pltpu.sync_copy(acc_sh.at[pl.ds(sid * SLAB, SLAB)],
                        out_hbm.at[cid, pl.ds(sid * SLAB, SLAB)])

    return deg_k(dst32)


def _agg_call(ya, yb, src16, dst16, d):
    @functools.partial(
        pl.kernel,
        out_type=jax.ShapeDtypeStruct((NC, NPAD, d), jnp.float32),
        mesh=plsc.VectorSubcoreMesh(**_MESH),
        scratch_types=[
            pltpu.VMEM((NCH_AGG, CH), jnp.int32),
            pltpu.VMEM((NCH_AGG, CH), jnp.int32),
            pltpu.VMEM((CH, d), jnp.float32),
            pltpu.VMEM((CH, d), jnp.float32),
            pltpu.VMEM_SHARED((NPAD, d), jnp.float32),
            pltpu.SemaphoreType.DMA,
            pltpu.SemaphoreType.DMA,
        ],
        compiler_params=pltpu.CompilerParams(use_tc_tiling_on_sc=False),
    )
    def agg_k(ya_hbm, yb_hbm, src_hbm, dst_hbm, out_hbm,
              src_v, dst_v, rows0, rows1, acc_sh, sem0, sem1):
        cid = lax.axis_index("c")
        sid = lax.axis_index("s")
        pltpu.sync_copy(src_hbm.at[sid], src_v)
        pltpu.sync_copy(dst_hbm.at[sid], dst_v)

        def run(tbl):
            pltpu.sync_copy(tbl.at[pl.ds(sid * SLAB, SLAB)],
                            acc_sh.at[pl.ds(sid * SLAB, SLAB)])
            plsc.subcore_barrier()
            pltpu.async_copy(tbl.at[src_v.at[0]], rows0, sem0)

            def body(g, carry):
                j0 = 2 * g
                j1 = j0 + 1
                pltpu.async_copy(tbl.at[src_v.at[j1]], rows1, sem1)
                pltpu.make_async_copy(tbl.at[src_v.at[j0]], rows0, sem0).wait()
                pltpu.sync_copy(rows0, acc_sh.at[dst_v.at[j0]], add=True)

                @pl.when(j0 + 2 < NCH_AGG)
                def _():
                    pltpu.async_copy(tbl.at[src_v.at[j0 + 2]], rows0, sem0)

                pltpu.make_async_copy(tbl.at[src_v.at[j1]], rows1, sem1).wait()
                pltpu.sync_copy(rows1, acc_sh.at[dst_v.at[j1]], add=True)
                return carry

            lax.fori_loop(0, NCH_AGG // 2, body, 0)

        @pl.when(cid == 0)
        def _():
            run(ya_hbm)

        @pl.when(cid == 1)
        def _():
            run(yb_hbm)

        plsc.subcore_barrier()
        pltpu.sync_copy(acc_sh.at[pl.ds(sid * SLAB, SLAB)],
                        out_hbm.at[cid, pl.ds(sid * SLAB, SLAB)])

    return agg_k(ya, yb, src16, dst16)


_R = 512


def _mm1_call(xpad, W1, deg):
    Q = HID_CH // 4

    def body(x_ref, w_ref, deg_ref, q0_ref, q1_ref, q2_ref, q3_ref, dinv_ref):
        d = deg_ref[0] + deg_ref[1] + 1.0
        dinv = lax.rsqrt(d)
        y = jnp.dot(x_ref[...], w_ref[...],
                    preferred_element_type=jnp.float32) * dinv
        q0_ref[...] = y[:, 0 * Q: 1 * Q]
        q1_ref[...] = y[:, 1 * Q: 2 * Q]
        q2_ref[...] = y[:, 2 * Q: 3 * Q]
        q3_ref[...] = y[:, 3 * Q: 4 * Q]
        dinv_ref[...] = dinv

    qspec = pl.BlockSpec((_R, Q), lambda i: (i, 0))
    qshape = jax.ShapeDtypeStruct((NPAD, Q), jnp.float32)
    return pl.pallas_call(
        body,
        grid=(NPAD // _R,),
        in_specs=[
            pl.BlockSpec((_R, IN_CH), lambda i: (i, 0)),
            pl.BlockSpec((IN_CH, HID_CH), lambda i: (0, 0)),
            pl.BlockSpec((NC, _R, 1), lambda i: (0, i, 0)),
        ],
        out_specs=[qspec, qspec, qspec, qspec,
                   pl.BlockSpec((_R, 1), lambda i: (i, 0))],
        out_shape=[qshape, qshape, qshape, qshape,
                   jax.ShapeDtypeStruct((NPAD, 1), jnp.float32)],
    )(xpad, W1, deg)


def _mm2_call(a0, a1, a2, a3, dinv, b1r, W2):
    Q = HID_CH // 4

    def body(a0_ref, a1_ref, a2_ref, a3_ref, dinv_ref, b1_ref, w_ref,
             ya_ref, yb_ref):
        dinv = dinv_ref[...]
        acc = jnp.concatenate(
            [a0_ref[...], a1_ref[...], a2_ref[...], a3_ref[...]], axis=1)
        h = jax.nn.relu(acc * dinv + b1_ref[...])
        y2 = jnp.dot(h, w_ref[...], preferred_element_type=jnp.float32) * dinv
        ya_ref[...] = y2[:, : OUT_CH // 2]
        yb_ref[...] = y2[:, OUT_CH // 2:]

    qspec = pl.BlockSpec((_R, Q), lambda i: (i, 0))
    return pl.pallas_call(
        body,
        grid=(NPAD // _R,),
        in_specs=[
            qspec, qspec, qspec, qspec,
            pl.BlockSpec((_R, 1), lambda i: (i, 0)),
            pl.BlockSpec((1, HID_CH), lambda i: (0, 0)),
            pl.BlockSpec((HID_CH, OUT_CH), lambda i: (0, 0)),
        ],
        out_specs=[
            pl.BlockSpec((_R, OUT_CH // 2), lambda i: (i, 0)),
            pl.BlockSpec((_R, OUT_CH // 2), lambda i: (i, 0)),
        ],
        out_shape=[
            jax.ShapeDtypeStruct((NPAD, OUT_CH // 2), jnp.float32),
            jax.ShapeDtypeStruct((NPAD, OUT_CH // 2), jnp.float32),
        ],
    )(a0, a1, a2, a3, dinv, b1r, W2)


def _fin_call(acc_a, acc_b, dinv, b2a, b2b):
    def body(aa_ref, ab_ref, dinv_ref, b2a_ref, b2b_ref, out_ref):
        dinv = dinv_ref[...]
        o1 = aa_ref[...] * dinv + b2a_ref[...]
        o2 = ab_ref[...] * dinv + b2b_ref[...]
        out_ref[...] = jnp.concatenate([o1, o2], axis=1)

    O2 = OUT_CH // 2
    return pl.pallas_call(
        body,
        grid=(NPAD // _R,),
        in_specs=[
            pl.BlockSpec((_R, O2), lambda i: (i, 0)),
            pl.BlockSpec((_R, O2), lambda i: (i, 0)),
            pl.BlockSpec((_R, 1), lambda i: (i, 0)),
            pl.BlockSpec((1, O2), lambda i: (0, 0)),
            pl.BlockSpec((1, O2), lambda i: (0, 0)),
        ],
        out_specs=pl.BlockSpec((_R, OUT_CH), lambda i: (i, 0)),
        out_shape=jax.ShapeDtypeStruct((NPAD, OUT_CH), jnp.float32),
    )(acc_a, acc_b, dinv, b2a, b2b)


def kernel(x, edge_index, W1, b1, W2, b2):
    ei = edge_index.astype(jnp.int32)
    src, dst = ei[0], ei[1]
    dst32 = dst.reshape(NC * NS, NCH_DEG, CH)
    src16 = src.reshape(NS, NCH_AGG, CH)
    dst16 = dst.reshape(NS, NCH_AGG, CH)
    xpad = jnp.pad(x, ((0, NPAD - N_NODES), (0, 0)))

    deg = _deg_call(dst32)
    q0, q1, q2, q3, dinv = _mm1_call(xpad, W1, deg.reshape(NC, NPAD, 1))
    accA = _agg_call(q0, q1, src16, dst16, HID_CH // 4)
    accB = _agg_call(q2, q3, src16, dst16, HID_CH // 4)
    y2a, y2b = _mm2_call(accA[0], accA[1], accB[0], accB[1], dinv,
                         b1.reshape(1, -1), W2)
    acc2 = _agg_call(y2a, y2b, src16, dst16, OUT_CH // 2)
    out = _fin_call(acc2[0], acc2[1], dinv,
                    b2[: OUT_CH // 2].reshape(1, -1),
                    b2[OUT_CH // 2:].reshape(1, -1))
    return out[:N_NODES]


# speedup vs baseline: 19.4081x; 1.0795x over previous
import functools

import jax
import jax.numpy as jnp
from jax import lax
from jax.experimental import pallas as pl
from jax.experimental.pallas import tpu as pltpu
from jax.experimental.pallas import tpu_sc as plsc

N_NODES = 10000
NPAD = 10240
IN_CH = 128
HID_CH = 256
OUT_CH = 128
N_EDGES = 320000

NC = 2
NS = 16
SLAB = NPAD // NS

CH = 80
NCH_DEG = N_EDGES // (NC * NS * CH)
CHA = 100
NCH_AGG = N_EDGES // (NS * CHA)
NBUF = 4

_MESH = dict(core_axis_name="c", subcore_axis_name="s")


def _deg_call(dst32):
    @functools.partial(
        pl.kernel,
        out_type=jax.ShapeDtypeStruct((NC, NPAD), jnp.float32),
        mesh=plsc.VectorSubcoreMesh(**_MESH),
        scratch_types=[
            pltpu.VMEM((NCH_DEG, CH), jnp.int32),
            pltpu.VMEM((CH,), jnp.float32),
            pltpu.VMEM((SLAB,), jnp.float32),
            pltpu.VMEM_SHARED((NPAD,), jnp.float32),
        ],
        compiler_params=pltpu.CompilerParams(use_tc_tiling_on_sc=False),
    )
    def deg_k(dst_hbm, out_hbm, idx_v, ones_v, z_v, acc_sh):
        cid = lax.axis_index("c")
        sid = lax.axis_index("s")
        w = cid * NS + sid
        for i in range(CH // 16):
            ones_v[pl.ds(i * 16, 16)] = jnp.ones((16,), jnp.float32)
        for i in range(SLAB // 16):
            z_v[pl.ds(i * 16, 16)] = jnp.zeros((16,), jnp.float32)
        pltpu.sync_copy(z_v, acc_sh.at[pl.ds(sid * SLAB, SLAB)])
        pltpu.sync_copy(dst_hbm.at[w], idx_v)
        plsc.subcore_barrier()

        def body(j, carry):
            pltpu.sync_copy(ones_v, acc_sh.at[idx_v.at[j]], add=True)
            return carry

        lax.fori_loop(0, NCH_DEG, body, 0)
        plsc.subcore_barrier()
        pltpu.sync_copy(acc_sh.at[pl.ds(sid * SLAB, SLAB)],
                        out_hbm.at[cid, pl.ds(sid * SLAB, SLAB)])

    return deg_k(dst32)


def _agg_call(ya, yb, src16, dst16, d):
    @functools.partial(
        pl.kernel,
        out_type=jax.ShapeDtypeStruct((NC, NPAD, d), jnp.float32),
        mesh=plsc.VectorSubcoreMesh(**_MESH),
        scratch_types=[
            pltpu.VMEM((NCH_AGG, CHA), jnp.int32),
            pltpu.VMEM((NCH_AGG, CHA), jnp.int32),
            pltpu.VMEM((CHA, d), jnp.float32),
            pltpu.VMEM((CHA, d), jnp.float32),
            pltpu.VMEM_SHARED((NPAD, d), jnp.float32),
            pltpu.SemaphoreType.DMA,
            pltpu.SemaphoreType.DMA,
        ],
        compiler_params=pltpu.CompilerParams(use_tc_tiling_on_sc=False),
    )
    def agg_k(ya_hbm, yb_hbm, src_hbm, dst_hbm, out_hbm,
              src_v, dst_v, rows0, rows1, acc_sh, sem0, sem1):
        cid = lax.axis_index("c")
        sid = lax.axis_index("s")
        pltpu.sync_copy(src_hbm.at[sid], src_v)
        pltpu.sync_copy(dst_hbm.at[sid], dst_v)

        def run(tbl):
            pltpu.sync_copy(tbl.at[pl.ds(sid * SLAB, SLAB)],
                            acc_sh.at[pl.ds(sid * SLAB, SLAB)])
            plsc.subcore_barrier()
            pltpu.async_copy(tbl.at[src_v.at[0]], rows0, sem0)

            def body(g, carry):
                j0 = 2 * g
                j1 = j0 + 1
                pltpu.async_copy(tbl.at[src_v.at[j1]], rows1, sem1)
                pltpu.make_async_copy(tbl.at[src_v.at[j0]], rows0, sem0).wait()
                pltpu.sync_copy(rows0, acc_sh.at[dst_v.at[j0]], add=True)

                @pl.when(j0 + 2 < NCH_AGG)
                def _():
                    pltpu.async_copy(tbl.at[src_v.at[j0 + 2]], rows0, sem0)

                pltpu.make_async_copy(tbl.at[src_v.at[j1]], rows1, sem1).wait()
                pltpu.sync_copy(rows1, acc_sh.at[dst_v.at[j1]], add=True)
                return carry

            lax.fori_loop(0, NCH_AGG // 2, body, 0)

        @pl.when(cid == 0)
        def _():
            run(ya_hbm)

        @pl.when(cid == 1)
        def _():
            run(yb_hbm)

        plsc.subcore_barrier()
        pltpu.sync_copy(acc_sh.at[pl.ds(sid * SLAB, SLAB)],
                        out_hbm.at[cid, pl.ds(sid * SLAB, SLAB)])

    return agg_k(ya, yb, src16, dst16)


_R = 512


def _mm1_call(xpad, W1, deg):
    Q = HID_CH // 4

    def body(x_ref, w_ref, deg_ref, q0_ref, q1_ref, q2_ref, q3_ref, dinv_ref):
        d = deg_ref[0] + deg_ref[1] + 1.0
        dinv = lax.rsqrt(d)
        y = jnp.dot(x_ref[...], w_ref[...],
                    preferred_element_type=jnp.float32) * dinv
        q0_ref[...] = y[:, 0 * Q: 1 * Q]
        q1_ref[...] = y[:, 1 * Q: 2 * Q]
        q2_ref[...] = y[:, 2 * Q: 3 * Q]
        q3_ref[...] = y[:, 3 * Q: 4 * Q]
        dinv_ref[...] = dinv

    qspec = pl.BlockSpec((_R, Q), lambda i: (i, 0))
    qshape = jax.ShapeDtypeStruct((NPAD, Q), jnp.float32)
    return pl.pallas_call(
        body,
        grid=(NPAD // _R,),
        in_specs=[
            pl.BlockSpec((_R, IN_CH), lambda i: (i, 0)),
            pl.BlockSpec((IN_CH, HID_CH), lambda i: (0, 0)),
            pl.BlockSpec((NC, _R, 1), lambda i: (0, i, 0)),
        ],
        out_specs=[qspec, qspec, qspec, qspec,
                   pl.BlockSpec((_R, 1), lambda i: (i, 0))],
        out_shape=[qshape, qshape, qshape, qshape,
                   jax.ShapeDtypeStruct((NPAD, 1), jnp.float32)],
    )(xpad, W1, deg)


def _mm2_call(a0, a1, a2, a3, dinv, b1r, W2):
    Q = HID_CH // 4

    def body(a0_ref, a1_ref, a2_ref, a3_ref, dinv_ref, b1_ref, w_ref,
             ya_ref, yb_ref):
        dinv = dinv_ref[...]
        acc = jnp.concatenate(
            [a0_ref[...], a1_ref[...], a2_ref[...], a3_ref[...]], axis=1)
        h = jax.nn.relu(acc * dinv + b1_ref[...])
        y2 = jnp.dot(h, w_ref[...], preferred_element_type=jnp.float32) * dinv
        ya_ref[...] = y2[:, : OUT_CH // 2]
        yb_ref[...] = y2[:, OUT_CH // 2:]

    qspec = pl.BlockSpec((_R, Q), lambda i: (i, 0))
    return pl.pallas_call(
        body,
        grid=(NPAD // _R,),
        in_specs=[
            qspec, qspec, qspec, qspec,
            pl.BlockSpec((_R, 1), lambda i: (i, 0)),
            pl.BlockSpec((1, HID_CH), lambda i: (0, 0)),
            pl.BlockSpec((HID_CH, OUT_CH), lambda i: (0, 0)),
        ],
        out_specs=[
            pl.BlockSpec((_R, OUT_CH // 2), lambda i: (i, 0)),
            pl.BlockSpec((_R, OUT_CH // 2), lambda i: (i, 0)),
        ],
        out_shape=[
            jax.ShapeDtypeStruct((NPAD, OUT_CH // 2), jnp.float32),
            jax.ShapeDtypeStruct((NPAD, OUT_CH // 2), jnp.float32),
        ],
    )(a0, a1, a2, a3, dinv, b1r, W2)


def _fin_call(acc_a, acc_b, dinv, b2a, b2b):
    def body(aa_ref, ab_ref, dinv_ref, b2a_ref, b2b_ref, out_ref):
        dinv = dinv_ref[...]
        o1 = aa_ref[...] * dinv + b2a_ref[...]
        o2 = ab_ref[...] * dinv + b2b_ref[...]
        out_ref[...] = jnp.concatenate([o1, o2], axis=1)

    O2 = OUT_CH // 2
    return pl.pallas_call(
        body,
        grid=(NPAD // _R,),
        in_specs=[
            pl.BlockSpec((_R, O2), lambda i: (i, 0)),
            pl.BlockSpec((_R, O2), lambda i: (i, 0)),
            pl.BlockSpec((_R, 1), lambda i: (i, 0)),
            pl.BlockSpec((1, O2), lambda i: (0, 0)),
            pl.BlockSpec((1, O2), lambda i: (0, 0)),
        ],
        out_specs=pl.BlockSpec((_R, OUT_CH), lambda i: (i, 0)),
        out_shape=jax.ShapeDtypeStruct((NPAD, OUT_CH), jnp.float32),
    )(acc_a, acc_b, dinv, b2a, b2b)


def kernel(x, edge_index, W1, b1, W2, b2):
    ei = edge_index.astype(jnp.int32)
    src, dst = ei[0], ei[1]
    dst32 = dst.reshape(NC * NS, NCH_DEG, CH)
    src16 = src.reshape(NS, NCH_AGG, CHA)
    dst16 = dst.reshape(NS, NCH_AGG, CHA)
    xpad = jnp.pad(x, ((0, NPAD - N_NODES), (0, 0)))

    deg = _deg_call(dst32)
    q0, q1, q2, q3, dinv = _mm1_call(xpad, W1, deg.reshape(NC, NPAD, 1))
    accA = _agg_call(q0, q1, src16, dst16, HID_CH // 4)
    accB = _agg_call(q2, q3, src16, dst16, HID_CH // 4)
    y2a, y2b = _mm2_call(accA[0], accA[1], accB[0], accB[1], dinv,
                         b1.reshape(1, -1), W2)
    acc2 = _agg_call(y2a, y2b, src16, dst16, OUT_CH // 2)
    out = _fin_call(acc2[0], acc2[1], dinv,
                    b2[: OUT_CH // 2].reshape(1, -1),
                    b2[OUT_CH // 2:].reshape(1, -1))
    return out[:N_NODES]


# speedup vs baseline: 20.8188x; 1.0727x over previous
import functools

import jax
import jax.numpy as jnp
from jax import lax
from jax.experimental import pallas as pl
from jax.experimental.pallas import tpu as pltpu
from jax.experimental.pallas import tpu_sc as plsc

N_NODES = 10000
NPAD = 10240
IN_CH = 128
HID_CH = 256
OUT_CH = 128
N_EDGES = 320000

NC = 2
NS = 16
SLAB = NPAD // NS

CH = 80
NCH_DEG = N_EDGES // (NC * NS * CH)
CHA = 125
NCH_AGG = N_EDGES // (NS * CHA)
NBUF = 4

_MESH = dict(core_axis_name="c", subcore_axis_name="s")


def _deg_call(dst32):
    @functools.partial(
        pl.kernel,
        out_type=jax.ShapeDtypeStruct((NC, NPAD), jnp.float32),
        mesh=plsc.VectorSubcoreMesh(**_MESH),
        scratch_types=[
            pltpu.VMEM((NCH_DEG, CH), jnp.int32),
            pltpu.VMEM((CH,), jnp.float32),
            pltpu.VMEM((SLAB,), jnp.float32),
            pltpu.VMEM_SHARED((NPAD,), jnp.float32),
        ],
        compiler_params=pltpu.CompilerParams(use_tc_tiling_on_sc=False),
    )
    def deg_k(dst_hbm, out_hbm, idx_v, ones_v, z_v, acc_sh):
        cid = lax.axis_index("c")
        sid = lax.axis_index("s")
        w = cid * NS + sid
        for i in range(CH // 16):
            ones_v[pl.ds(i * 16, 16)] = jnp.ones((16,), jnp.float32)
        for i in range(SLAB // 16):
            z_v[pl.ds(i * 16, 16)] = jnp.zeros((16,), jnp.float32)
        pltpu.sync_copy(z_v, acc_sh.at[pl.ds(sid * SLAB, SLAB)])
        pltpu.sync_copy(dst_hbm.at[w], idx_v)
        plsc.subcore_barrier()

        def body(j, carry):
            pltpu.sync_copy(ones_v, acc_sh.at[idx_v.at[j]], add=True)
            return carry

        lax.fori_loop(0, NCH_DEG, body, 0)
        plsc.subcore_barrier()
        pltpu.sync_copy(acc_sh.at[pl.ds(sid * SLAB, SLAB)],
                        out_hbm.at[cid, pl.ds(sid * SLAB, SLAB)])

    return deg_k(dst32)


def _agg_call(ya, yb, src16, dst16, d):
    @functools.partial(
        pl.kernel,
        out_type=jax.ShapeDtypeStruct((NC, NPAD, d), jnp.float32),
        mesh=plsc.VectorSubcoreMesh(**_MESH),
        scratch_types=[
            pltpu.VMEM((NCH_AGG, CHA), jnp.int32),
            pltpu.VMEM((NCH_AGG, CHA), jnp.int32),
            pltpu.VMEM((CHA, d), jnp.float32),
            pltpu.VMEM((CHA, d), jnp.float32),
            pltpu.VMEM_SHARED((NPAD, d), jnp.float32),
            pltpu.SemaphoreType.DMA,
            pltpu.SemaphoreType.DMA,
        ],
        compiler_params=pltpu.CompilerParams(use_tc_tiling_on_sc=False),
    )
    def agg_k(ya_hbm, yb_hbm, src_hbm, dst_hbm, out_hbm,
              src_v, dst_v, rows0, rows1, acc_sh, sem0, sem1):
        cid = lax.axis_index("c")
        sid = lax.axis_index("s")
        pltpu.sync_copy(src_hbm.at[sid], src_v)
        pltpu.sync_copy(dst_hbm.at[sid], dst_v)

        def run(tbl):
            pltpu.sync_copy(tbl.at[pl.ds(sid * SLAB, SLAB)],
                            acc_sh.at[pl.ds(sid * SLAB, SLAB)])
            plsc.subcore_barrier()
            pltpu.async_copy(tbl.at[src_v.at[0]], rows0, sem0)

            def body(g, carry):
                j0 = 2 * g
                j1 = j0 + 1
                pltpu.async_copy(tbl.at[src_v.at[j1]], rows1, sem1)
                pltpu.make_async_copy(tbl.at[src_v.at[j0]], rows0, sem0).wait()
                pltpu.sync_copy(rows0, acc_sh.at[dst_v.at[j0]], add=True)

                @pl.when(j0 + 2 < NCH_AGG)
                def _():
                    pltpu.async_copy(tbl.at[src_v.at[j0 + 2]], rows0, sem0)

                pltpu.make_async_copy(tbl.at[src_v.at[j1]], rows1, sem1).wait()
                pltpu.sync_copy(rows1, acc_sh.at[dst_v.at[j1]], add=True)
                return carry

            lax.fori_loop(0, NCH_AGG // 2, body, 0)

        @pl.when(cid == 0)
        def _():
            run(ya_hbm)

        @pl.when(cid == 1)
        def _():
            run(yb_hbm)

        plsc.subcore_barrier()
        pltpu.sync_copy(acc_sh.at[pl.ds(sid * SLAB, SLAB)],
                        out_hbm.at[cid, pl.ds(sid * SLAB, SLAB)])

    return agg_k(ya, yb, src16, dst16)


_R = 512


def _mm1_call(xpad, W1, deg):
    Q = HID_CH // 4

    def body(x_ref, w_ref, deg_ref, q0_ref, q1_ref, q2_ref, q3_ref, dinv_ref):
        d = deg_ref[0] + deg_ref[1] + 1.0
        dinv = lax.rsqrt(d)
        y = jnp.dot(x_ref[...], w_ref[...],
                    preferred_element_type=jnp.float32) * dinv
        q0_ref[...] = y[:, 0 * Q: 1 * Q]
        q1_ref[...] = y[:, 1 * Q: 2 * Q]
        q2_ref[...] = y[:, 2 * Q: 3 * Q]
        q3_ref[...] = y[:, 3 * Q: 4 * Q]
        dinv_ref[...] = dinv

    qspec = pl.BlockSpec((_R, Q), lambda i: (i, 0))
    qshape = jax.ShapeDtypeStruct((NPAD, Q), jnp.float32)
    return pl.pallas_call(
        body,
        grid=(NPAD // _R,),
        in_specs=[
            pl.BlockSpec((_R, IN_CH), lambda i: (i, 0)),
            pl.BlockSpec((IN_CH, HID_CH), lambda i: (0, 0)),
            pl.BlockSpec((NC, _R, 1), lambda i: (0, i, 0)),
        ],
        out_specs=[qspec, qspec, qspec, qspec,
                   pl.BlockSpec((_R, 1), lambda i: (i, 0))],
        out_shape=[qshape, qshape, qshape, qshape,
                   jax.ShapeDtypeStruct((NPAD, 1), jnp.float32)],
    )(xpad, W1, deg)


def _mm2_call(a0, a1, a2, a3, dinv, b1r, W2):
    Q = HID_CH // 4

    def body(a0_ref, a1_ref, a2_ref, a3_ref, dinv_ref, b1_ref, w_ref,
             ya_ref, yb_ref):
        dinv = dinv_ref[...]
        acc = jnp.concatenate(
            [a0_ref[...], a1_ref[...], a2_ref[...], a3_ref[...]], axis=1)
        h = jax.nn.relu(acc * dinv + b1_ref[...])
        y2 = jnp.dot(h, w_ref[...], preferred_element_type=jnp.float32) * dinv
        ya_ref[...] = y2[:, : OUT_CH // 2]
        yb_ref[...] = y2[:, OUT_CH // 2:]

    qspec = pl.BlockSpec((_R, Q), lambda i: (i, 0))
    return pl.pallas_call(
        body,
        grid=(NPAD // _R,),
        in_specs=[
            qspec, qspec, qspec, qspec,
            pl.BlockSpec((_R, 1), lambda i: (i, 0)),
            pl.BlockSpec((1, HID_CH), lambda i: (0, 0)),
            pl.BlockSpec((HID_CH, OUT_CH), lambda i: (0, 0)),
        ],
        out_specs=[
            pl.BlockSpec((_R, OUT_CH // 2), lambda i: (i, 0)),
            pl.BlockSpec((_R, OUT_CH // 2), lambda i: (i, 0)),
        ],
        out_shape=[
            jax.ShapeDtypeStruct((NPAD, OUT_CH // 2), jnp.float32),
            jax.ShapeDtypeStruct((NPAD, OUT_CH // 2), jnp.float32),
        ],
    )(a0, a1, a2, a3, dinv, b1r, W2)


def _fin_call(acc_a, acc_b, dinv, b2a, b2b):
    def body(aa_ref, ab_ref, dinv_ref, b2a_ref, b2b_ref, out_ref):
        dinv = dinv_ref[...]
        o1 = aa_ref[...] * dinv + b2a_ref[...]
        o2 = ab_ref[...] * dinv + b2b_ref[...]
        out_ref[...] = jnp.concatenate([o1, o2], axis=1)

    O2 = OUT_CH // 2
    return pl.pallas_call(
        body,
        grid=(NPAD // _R,),
        in_specs=[
            pl.BlockSpec((_R, O2), lambda i: (i, 0)),
            pl.BlockSpec((_R, O2), lambda i: (i, 0)),
            pl.BlockSpec((_R, 1), lambda i: (i, 0)),
            pl.BlockSpec((1, O2), lambda i: (0, 0)),
            pl.BlockSpec((1, O2), lambda i: (0, 0)),
        ],
        out_specs=pl.BlockSpec((_R, OUT_CH), lambda i: (i, 0)),
        out_shape=jax.ShapeDtypeStruct((NPAD, OUT_CH), jnp.float32),
    )(acc_a, acc_b, dinv, b2a, b2b)


def kernel(x, edge_index, W1, b1, W2, b2):
    ei = edge_index.astype(jnp.int32)
    src, dst = ei[0], ei[1]
    dst32 = dst.reshape(NC * NS, NCH_DEG, CH)
    src16 = src.reshape(NS, NCH_AGG, CHA)
    dst16 = dst.reshape(NS, NCH_AGG, CHA)
    xpad = jnp.pad(x, ((0, NPAD - N_NODES), (0, 0)))

    deg = _deg_call(dst32)
    q0, q1, q2, q3, dinv = _mm1_call(xpad, W1, deg.reshape(NC, NPAD, 1))
    accA = _agg_call(q0, q1, src16, dst16, HID_CH // 4)
    accB = _agg_call(q2, q3, src16, dst16, HID_CH // 4)
    y2a, y2b = _mm2_call(accA[0], accA[1], accB[0], accB[1], dinv,
                         b1.reshape(1, -1), W2)
    acc2 = _agg_call(y2a, y2b, src16, dst16, OUT_CH // 2)
    out = _fin_call(acc2[0], acc2[1], dinv,
                    b2[: OUT_CH // 2].reshape(1, -1),
                    b2[OUT_CH // 2:].reshape(1, -1))
    return out[:N_NODES]
